# baseline (speedup 1.0000x reference)
import functools

import jax
import jax.numpy as jnp
from jax import lax
from jax.experimental import pallas as pl
from jax.experimental.pallas import tpu as pltpu
from jax.experimental.pallas import tpu_sc as plsc

_L = 128
_D = 64
_DP = 64


@functools.lru_cache(maxsize=None)
def _build(n_hist, n_blocks, vocab):
    info = plsc.get_sparse_core_info()
    nw = info.num_cores * info.num_subcores
    assert n_blocks == nw and n_hist % 4 == 0

    mesh = plsc.VectorSubcoreMesh(core_axis_name="c", subcore_axis_name="s")

    @functools.partial(
        pl.kernel,
        mesh=mesh,
        compiler_params=pltpu.CompilerParams(
            use_tc_tiling_on_sc=False, needs_layout_passes=False),
        out_type=jax.ShapeDtypeStruct((n_hist, 8, n_blocks, 8 * _L), jnp.float32),
        scratch_types=[
            pltpu.VMEM((n_hist, _L), jnp.int32),
            pltpu.VMEM((4, _L, _DP), jnp.float32),
            pltpu.VMEM((2, 8, 8 * _L), jnp.float32),
            pltpu.SemaphoreType.DMA,
            pltpu.SemaphoreType.DMA,
            pltpu.SemaphoreType.DMA,
            pltpu.SemaphoreType.DMA,
            pltpu.SemaphoreType.DMA,
            pltpu.SemaphoreType.DMA,
        ],
    )
    def emb_gather(table_hbm, idx_hbm, out_hbm, idx_v, gbuf, obuf,
                   gsem0, gsem1, gsem2, gsem3, wsem0, wsem1):
        w = lax.axis_index("s") * info.num_cores + lax.axis_index("c")
        gsems = (gsem0, gsem1, gsem2, gsem3)
        wsems = (wsem0, wsem1)

        pltpu.sync_copy(idx_hbm.at[:, w], idx_v)

        rowidx = [jnp.full((16,), bb * 16, jnp.int32) + lax.iota(jnp.int32, 16)
                  for bb in range(8)]

        def fire_gather(h, s):
            pltpu.async_copy(table_hbm.at[idx_v.at[h]], gbuf.at[s], gsems[s])

        def wait_gather(h, s):
            pltpu.make_async_copy(
                table_hbm.at[idx_v.at[h]], gbuf.at[s], gsems[s]).wait()

        def fire_write(h, os):
            pltpu.async_copy(obuf.at[os], out_hbm.at[h, :, w], wsems[os])

        def wait_write(os):
            pltpu.make_async_copy(
                obuf.at[os], out_hbm.at[0, :, w], wsems[os]).wait()

        def transpose(s, os):
            def dbody(dt, c):
                for dr in range(8):
                    col = jnp.broadcast_to(dt * 8 + dr, (16,))
                    for bb in range(8):
                        v = plsc.load_gather(gbuf.at[s], [rowidx[bb], col])
                        obuf[os, dt, pl.ds(dr * _L + bb * 16, 16)] = v
                return c
            lax.fori_loop(0, 8, dbody, 0, unroll=False)

        def process(h, b):
            s, os = b, b % 2
            wait_gather(h, s)
            wait_write(os)
            transpose(s, os)
            fire_write(h, os)

        fire_gather(0, 0)
        fire_gather(1, 1)
        fire_gather(2, 2)
        fire_write(0, 0)
        fire_write(1, 1)

        def body(k, c):
            h = k * 4
            for b in range(4):
                process(h + b, b)
                fire_gather(h + b + 3, (b + 3) % 4)
            return c

        lax.fori_loop(0, (n_hist - 4) // 4, body, 0, unroll=False)

        fire_gather(n_hist - 1, 3)
        for b in range(4):
            process(n_hist - 4 + b, b)
        wait_write(0)
        wait_write(1)

    return emb_gather


def kernel(input_variable, emb_weight):
    batch, hist = input_variable.shape
    vocab, dim = emb_weight.shape
    n_blocks = batch // _L
    idx3 = input_variable.T.reshape(hist, n_blocks, _L)
    if idx3.dtype != jnp.int32:
        idx3 = idx3.astype(jnp.int32)
    p = _build(hist, n_blocks, vocab)(emb_weight, idx3)
    p5 = p.reshape(hist, 8, n_blocks, 8, _L)
    return p5.transpose(2, 4, 0, 1, 3).reshape(batch, hist, dim)
